# baseline (speedup 1.0000x reference)
import functools

import jax
import jax.numpy as jnp
from jax import lax
from jax.experimental import pallas as pl
from jax.experimental.pallas import tpu as pltpu
from jax.experimental.pallas import tpu_sc as plsc

_VOCAB = 100000
_DIM = 128
_B = 4096
_L = 50
_N = _B * _L
_NW = 32
_PER_W = _N // _NW
_CHUNK = 64
_NCHUNK = _PER_W // _CHUNK
_NBUF = 4
_ROUNDS = _NCHUNK // _NBUF - 1

_mesh = plsc.VectorSubcoreMesh(core_axis_name="c", subcore_axis_name="s")


@functools.partial(
    pl.kernel,
    mesh=_mesh,
    out_type=(
        jax.ShapeDtypeStruct((_N, _DIM), jnp.float32),
        jax.ShapeDtypeStruct((_N, _DIM), jnp.float32),
    ),
    scratch_types=[
        pltpu.VMEM((_PER_W,), jnp.int32),
        pltpu.VMEM((_NBUF, _CHUNK, _DIM), jnp.float32),
        pltpu.VMEM((_NBUF, _CHUNK, _DIM), jnp.float32),
        pltpu.SemaphoreType.DMA((_NBUF,)),
        pltpu.SemaphoreType.DMA((_NBUF,)),
    ],
)
def _gather2(x_hbm, wr_hbm, wi_hbm, outr_hbm, outi_hbm,
             idx_v, bufr, bufi, sem_in, sem_out):
    wid = lax.axis_index("s") * 2 + lax.axis_index("c")
    base = wid * _PER_W
    pltpu.sync_copy(x_hbm.at[pl.ds(base, _PER_W)], idx_v)

    def start_gather(c, b):
        idx = idx_v.at[pl.ds(c * _CHUNK, _CHUNK)]
        pltpu.async_copy(wr_hbm.at[idx], bufr.at[b], sem_in.at[b])
        pltpu.async_copy(wi_hbm.at[idx], bufi.at[b], sem_in.at[b])

    def wait_gather(b):
        pltpu.make_async_copy(wr_hbm.at[pl.ds(0, _CHUNK)], bufr.at[b],
                              sem_in.at[b]).wait()
        pltpu.make_async_copy(wi_hbm.at[pl.ds(0, _CHUNK)], bufi.at[b],
                              sem_in.at[b]).wait()

    def start_out(c, b):
        dst = pl.ds(base + c * _CHUNK, _CHUNK)
        pltpu.async_copy(bufr.at[b], outr_hbm.at[dst], sem_out.at[b])
        pltpu.async_copy(bufi.at[b], outi_hbm.at[dst], sem_out.at[b])

    def wait_out(c, b):
        dst = pl.ds(base + c * _CHUNK, _CHUNK)
        pltpu.make_async_copy(bufr.at[b], outr_hbm.at[dst],
                              sem_out.at[b]).wait()
        pltpu.make_async_copy(bufi.at[b], outi_hbm.at[dst],
                              sem_out.at[b]).wait()

    K = 2

    def step(c, b, refill, drain_first):
        wait_gather(b)
        start_out(c, b)
        if refill:
            bk = (b + K) % _NBUF
            if drain_first:
                wait_out(c + K - _NBUF, bk)
            start_gather(c + K, bk)

    for b in range(K):
        start_gather(b, b)
    for c in range(_NBUF - K):
        step(c, c % _NBUF, True, False)

    def round_body(g, carry):
        for b in range(_NBUF):
            c = (g + 1) * _NBUF + b - K
            step(c, (b + _NBUF - K) % _NBUF, True, True)
        return carry

    lax.fori_loop(0, (_NCHUNK - _NBUF) // _NBUF, round_body, 0)

    for c in range(_NCHUNK - K, _NCHUNK):
        step(c, c % _NBUF, False, False)
    for c in range(_NCHUNK - _NBUF, _NCHUNK):
        wait_out(c, c % _NBUF)


@jax.jit
def kernel(x, W_real, W_imag):
    xt = x.T.reshape(_N)
    real, imag = _gather2(xt, W_real, W_imag)
    real3 = real.reshape(_L, _B, _DIM).transpose(1, 0, 2)
    imag3 = imag.reshape(_L, _B, _DIM).transpose(1, 0, 2)
    return lax.complex(real3, imag3)
